# Initial kernel scaffold; baseline (speedup 1.0000x reference)
import jax
import jax.numpy as jnp
from jax.experimental import pallas as pl


def kernel(feats, y, conf):
    raise NotImplementedError("write your pallas kernel here")


# speedup vs baseline: 6.4266x; 6.4266x over previous
import functools

import jax
import jax.numpy as jnp
from jax.experimental import pallas as pl
from jax.experimental.pallas import tpu as pltpu

_K = 1000
_KP = 1024
_D = 2048
_ALPHA = 10.0
_TOPM = 20
_BR = 512


def _segsum_body(y_ref, f_ref, sums_ref, cnts_ref):
    i = pl.program_id(0)
    yv = y_ref[0]
    rows = jax.lax.broadcasted_iota(jnp.int32, (_KP, _BR), 0)
    oh = (rows == yv).astype(jnp.float32)
    f = f_ref[...]
    part = jax.lax.dot_general(
        oh.astype(jnp.bfloat16), f.astype(jnp.bfloat16),
        (((1,), (0,)), ((), ())),
        preferred_element_type=jnp.float32)
    c = jnp.sum(oh, axis=1, keepdims=True)

    @pl.when(i == 0)
    def _init():
        sums_ref[...] = part
        cnts_ref[...] = c

    @pl.when(i > 0)
    def _acc():
        sums_ref[...] = sums_ref[...] + part
        cnts_ref[...] = cnts_ref[...] + c


def _loss_body(sums_ref, cnts_ref, cnts_t_ref, conf_ref, conf_t_ref, out_ref,
               key_ref, msk_ref):
    s = sums_ref[...]
    n2 = jnp.sum(s * s, axis=1, keepdims=True)
    inv = jnp.where(n2 > 0.0, jax.lax.rsqrt(jnp.maximum(n2, 1e-30)), 0.0)
    sn = s * inv
    g = jax.lax.dot_general(
        sn, sn, (((1,), (1,)), ((), ())),
        preferred_element_type=jnp.float32)

    row_i = jax.lax.broadcasted_iota(jnp.int32, (_KP, _KP), 0)
    col_i = jax.lax.broadcasted_iota(jnp.int32, (_KP, _KP), 1)
    eye = (row_i == col_i)

    zr = jnp.sum(jnp.where(eye, g, 0.0), axis=1, keepdims=True)
    zc = jnp.sum(jnp.where(eye, g, 0.0), axis=0, keepdims=True)
    d2 = zr + zc - 2.0 * g

    omega = 0.5 * (conf_ref[...] + conf_t_ref[...])
    omega = jnp.where(eye, 0.0, omega)

    pr = (cnts_ref[...] > 0.0).astype(jnp.float32)
    pc = (cnts_t_ref[...] > 0.0).astype(jnp.float32)
    csize = jnp.sum(pr)
    ppair = pr * pc
    pairmask = ppair * (row_i < col_i).astype(jnp.float32)

    contrib = pairmask * omega * jnp.exp(-_ALPHA * d2)
    num_all = jnp.sum(contrib)

    scores = jnp.where(ppair > 0.0, omega, -1.0)
    si = jnp.round(jnp.maximum(scores, 0.0) * 16777216.0).astype(jnp.int32)
    si = jnp.where(scores < 0.0, jnp.int32(-1), si)
    key_ref[...] = si * 1024 + (1023 - col_i)
    msk_ref[...] = jnp.zeros((_KP, _KP), jnp.float32)

    def body(_, carry):
        k = key_ref[...]
        mx = jnp.max(k, axis=1, keepdims=True)
        sel = k == mx
        msk_ref[...] = jnp.where(sel, 1.0, msk_ref[...])
        key_ref[...] = jnp.where(sel, jnp.int32(-(2 ** 31 - 1)), k)
        return carry

    jax.lax.fori_loop(0, _TOPM, body, jnp.int32(0))
    mskf = msk_ref[...]
    num_tk = jnp.sum(mskf * contrib)
    den_tk = jnp.sum(mskf * pairmask)

    loss_all = num_all / jnp.maximum(csize * (csize - 1.0) * 0.5, 1.0)
    loss_tk = num_tk / jnp.maximum(den_tk, 1.0)
    loss = jnp.where(csize < 2.0, 0.0,
                     jnp.where(csize <= float(_TOPM + 1), loss_all, loss_tk))
    out_ref[...] = jnp.broadcast_to(loss, (1, 1))


@jax.jit
def kernel(feats, y, conf):
    b = feats.shape[0]
    nb = b // _BR
    y3 = y.astype(jnp.int32).reshape(nb, 1, _BR)
    conf_p = jnp.pad(conf, ((0, _KP - _K), (0, _KP - _K)))
    conf_t = conf_p.T

    sums, cnts = pl.pallas_call(
        _segsum_body,
        grid=(nb,),
        in_specs=[
            pl.BlockSpec((1, 1, _BR), lambda i: (i, 0, 0)),
            pl.BlockSpec((_BR, _D), lambda i: (i, 0)),
        ],
        out_specs=[
            pl.BlockSpec((_KP, _D), lambda i: (0, 0)),
            pl.BlockSpec((_KP, 1), lambda i: (0, 0)),
        ],
        out_shape=[
            jax.ShapeDtypeStruct((_KP, _D), jnp.float32),
            jax.ShapeDtypeStruct((_KP, 1), jnp.float32),
        ],
    )(y3, feats)

    cnts_t = cnts.reshape(1, _KP)

    out = pl.pallas_call(
        _loss_body,
        out_shape=jax.ShapeDtypeStruct((1, 1), jnp.float32),
        scratch_shapes=[
            pltpu.VMEM((_KP, _KP), jnp.int32),
            pltpu.VMEM((_KP, _KP), jnp.float32),
        ],
    )(sums, cnts, cnts_t, conf_p, conf_t)
    return out[0, 0]


# speedup vs baseline: 6.9858x; 1.0870x over previous
import jax
import jax.numpy as jnp
from jax.experimental import pallas as pl
from jax.experimental.pallas import tpu as pltpu

_K = 1000
_KP = 1024
_D = 2048
_ALPHA = 10.0
_TOPM = 20
_BR = 1024


def _segsum_body(y_ref, f_ref, sums_ref):
    i = pl.program_id(0)
    yv = y_ref[0]
    rows = jax.lax.broadcasted_iota(jnp.int32, (_KP, _BR), 0)
    oh = (rows == yv).astype(jnp.float32)
    part = jax.lax.dot_general(
        oh.astype(jnp.bfloat16), f_ref[...].astype(jnp.bfloat16),
        (((1,), (0,)), ((), ())),
        preferred_element_type=jnp.float32)

    @pl.when(i == 0)
    def _init():
        sums_ref[...] = part

    @pl.when(i > 0)
    def _acc():
        sums_ref[...] = sums_ref[...] + part


def _loss_body(sums_ref, conf_ref, conf_t_ref, out_ref, key_ref, msk_ref):
    s = sums_ref[...]
    n2 = jnp.sum(s * s, axis=1, keepdims=True)
    inv = jnp.where(n2 > 0.0, jax.lax.rsqrt(jnp.maximum(n2, 1e-30)), 0.0)
    sn = (s * inv).astype(jnp.bfloat16)
    g = jax.lax.dot_general(
        sn, sn, (((1,), (1,)), ((), ())),
        preferred_element_type=jnp.float32)

    row_i = jax.lax.broadcasted_iota(jnp.int32, (_KP, _KP), 0)
    col_i = jax.lax.broadcasted_iota(jnp.int32, (_KP, _KP), 1)
    eye = row_i == col_i
    gd = jnp.where(eye, g, 0.0)
    zr = jnp.sum(gd, axis=1, keepdims=True)
    zc = jnp.sum(gd, axis=0, keepdims=True)
    d2 = zr + zc - 2.0 * g

    pr = (zr > 0.5).astype(jnp.float32)
    pc = (zc > 0.5).astype(jnp.float32)
    csize = jnp.sum(pr)
    ppair = pr * pc

    omega = 0.5 * (conf_ref[...] + conf_t_ref[...])
    omega = jnp.where(eye, 0.0, omega)

    pairmask = ppair * (row_i > col_i).astype(jnp.float32)
    contrib = pairmask * omega * jnp.exp(-_ALPHA * d2)
    num_all = jnp.sum(contrib)

    scores = jnp.where(ppair > 0.5, omega, -1.0)
    si = jnp.round(jnp.maximum(scores, 0.0) * 16777216.0).astype(jnp.int32)
    si = jnp.where(scores < 0.0, jnp.int32(-1), si)
    key_ref[...] = si * 1024 + (1023 - row_i)
    msk_ref[...] = jnp.zeros((_KP, _KP), jnp.float32)

    def body(_, carry):
        k = key_ref[...]
        mx = jnp.max(k, axis=0, keepdims=True)
        sel = k == mx
        msk_ref[...] = jnp.where(sel, 1.0, msk_ref[...])
        key_ref[...] = jnp.where(sel, jnp.int32(-(2 ** 31 - 1)), k)
        return carry

    jax.lax.fori_loop(0, _TOPM, body, jnp.int32(0))
    mskf = msk_ref[...]
    num_tk = jnp.sum(mskf * contrib)
    den_tk = jnp.sum(mskf * pairmask)

    loss_all = num_all / jnp.maximum(csize * (csize - 1.0) * 0.5, 1.0)
    loss_tk = num_tk / jnp.maximum(den_tk, 1.0)
    loss = jnp.where(csize < 1.5, 0.0,
                     jnp.where(csize <= float(_TOPM + 1) + 0.5,
                               loss_all, loss_tk))
    out_ref[...] = jnp.broadcast_to(loss, (1, 1))


@jax.jit
def kernel(feats, y, conf):
    b = feats.shape[0]
    nb = b // _BR
    y3 = y.astype(jnp.int32).reshape(nb, 1, _BR)
    conf_p = jnp.pad(conf, ((0, _KP - _K), (0, _KP - _K)))
    conf_t = conf_p.T

    sums = pl.pallas_call(
        _segsum_body,
        grid=(nb,),
        in_specs=[
            pl.BlockSpec((1, 1, _BR), lambda i: (i, 0, 0)),
            pl.BlockSpec((_BR, _D), lambda i: (i, 0)),
        ],
        out_specs=pl.BlockSpec((_KP, _D), lambda i: (0, 0)),
        out_shape=jax.ShapeDtypeStruct((_KP, _D), jnp.float32),
    )(y3, feats)

    out = pl.pallas_call(
        _loss_body,
        out_shape=jax.ShapeDtypeStruct((1, 1), jnp.float32),
        scratch_shapes=[
            pltpu.VMEM((_KP, _KP), jnp.int32),
            pltpu.VMEM((_KP, _KP), jnp.float32),
        ],
    )(sums, conf_p, conf_t)
    return out[0, 0]


# speedup vs baseline: 8.2458x; 1.1804x over previous
import jax
import jax.numpy as jnp
from jax.experimental import pallas as pl
from jax.experimental.pallas import tpu as pltpu

_K = 1000
_KP = 1024
_D = 2048
_ALPHA = 10.0
_TOPM = 20
_BR = 2048


def _segsum_body(y_ref, f_ref, sums_ref):
    i = pl.program_id(0)
    yv = y_ref[0]
    rows = jax.lax.broadcasted_iota(jnp.int32, (_KP, _BR), 0)
    oh = (rows == yv).astype(jnp.float32)
    part = jax.lax.dot_general(
        oh, f_ref[...],
        (((1,), (0,)), ((), ())),
        precision=jax.lax.Precision.DEFAULT,
        preferred_element_type=jnp.float32)

    @pl.when(i == 0)
    def _init():
        sums_ref[...] = part.astype(jnp.bfloat16)

    @pl.when(i > 0)
    def _acc():
        sums_ref[...] = (sums_ref[...].astype(jnp.float32)
                         + part).astype(jnp.bfloat16)


def _loss_body(sums_ref, conf_ref, conf_t_ref, out_ref, key_ref):
    s = sums_ref[...].astype(jnp.float32)
    n2 = jnp.sum(s * s, axis=1, keepdims=True)
    inv = jnp.where(n2 > 0.0, jax.lax.rsqrt(jnp.maximum(n2, 1e-30)), 0.0)
    sn = (s * inv).astype(jnp.bfloat16)
    g = jax.lax.dot_general(
        sn, sn, (((1,), (1,)), ((), ())),
        preferred_element_type=jnp.float32)

    row_i = jax.lax.broadcasted_iota(jnp.int32, (_KP, _KP), 0)
    col_i = jax.lax.broadcasted_iota(jnp.int32, (_KP, _KP), 1)
    eye = row_i == col_i
    gd = jnp.where(eye, g, 0.0)
    zr = jnp.sum(gd, axis=1, keepdims=True)
    zc = jnp.sum(gd, axis=0, keepdims=True)
    d2 = zr + zc - 2.0 * g

    pr = (zr > 0.5).astype(jnp.float32)
    pc = (zc > 0.5).astype(jnp.float32)
    csize = jnp.sum(pr)
    ppair = pr * pc

    omega = 0.5 * (conf_ref[...] + conf_t_ref[...])
    omega = jnp.where(eye, 0.0, omega)

    pairmask = ppair * (row_i > col_i).astype(jnp.float32)
    contrib = pairmask * omega * jnp.exp(-_ALPHA * d2)
    num_all = jnp.sum(contrib)

    scores = jnp.where(ppair > 0.5, omega, -1.0)
    si = jnp.round(jnp.maximum(scores, 0.0) * 16777216.0).astype(jnp.int32)
    si = jnp.where(scores < 0.0, jnp.int32(-1), si)
    key_ref[...] = si * 1024 + (1023 - row_i)

    taken = jnp.int32(-(2 ** 31 - 1))

    def body(_, carry):
        k = key_ref[...]
        mx = jnp.max(k, axis=0, keepdims=True)
        key_ref[...] = jnp.where(k == mx, taken, k)
        return carry

    jax.lax.fori_loop(0, _TOPM, body, jnp.int32(0))
    mskf = (key_ref[...] == taken).astype(jnp.float32)
    num_tk = jnp.sum(mskf * contrib)
    den_tk = jnp.sum(mskf * pairmask)

    loss_all = num_all / jnp.maximum(csize * (csize - 1.0) * 0.5, 1.0)
    loss_tk = num_tk / jnp.maximum(den_tk, 1.0)
    loss = jnp.where(csize < 1.5, 0.0,
                     jnp.where(csize <= float(_TOPM + 1) + 0.5,
                               loss_all, loss_tk))
    out_ref[...] = jnp.broadcast_to(loss, (1, 1))


@jax.jit
def kernel(feats, y, conf):
    b = feats.shape[0]
    nb = b // _BR
    y3 = y.astype(jnp.int32).reshape(nb, 1, _BR)
    conf_p = jnp.pad(conf, ((0, _KP - _K), (0, _KP - _K)))
    conf_t = conf_p.T

    sums = pl.pallas_call(
        _segsum_body,
        grid=(nb,),
        in_specs=[
            pl.BlockSpec((1, 1, _BR), lambda i: (i, 0, 0)),
            pl.BlockSpec((_BR, _D), lambda i: (i, 0)),
        ],
        out_specs=pl.BlockSpec((_KP, _D), lambda i: (0, 0)),
        out_shape=jax.ShapeDtypeStruct((_KP, _D), jnp.bfloat16),
    )(y3, feats)

    out = pl.pallas_call(
        _loss_body,
        out_shape=jax.ShapeDtypeStruct((1, 1), jnp.float32),
        scratch_shapes=[
            pltpu.VMEM((_KP, _KP), jnp.int32),
        ],
    )(sums, conf_p, conf_t)
    return out[0, 0]


# speedup vs baseline: 9.2602x; 1.1230x over previous
import jax
import jax.numpy as jnp
from jax.experimental import pallas as pl
from jax.experimental.pallas import tpu as pltpu

_K = 1000
_KP = 1024
_D = 2048
_ALPHA = 10.0
_TOPM = 20
_BR = 2048
_NB = 16384 // _BR


def _body(y_ref, f_ref, conf_ref, out_ref, sums_ref, key_ref):
    i = pl.program_id(0)

    @pl.when(i < _NB)
    def _segsum():
        yv = y_ref[0]
        rows = jax.lax.broadcasted_iota(jnp.int32, (_KP, _BR), 0)
        oh = (rows == yv).astype(jnp.float32)
        part = jax.lax.dot_general(
            oh, f_ref[...],
            (((1,), (0,)), ((), ())),
            precision=jax.lax.Precision.DEFAULT,
            preferred_element_type=jnp.float32)

        @pl.when(i == 0)
        def _init():
            sums_ref[...] = part.astype(jnp.bfloat16)

        @pl.when(i > 0)
        def _acc():
            sums_ref[...] = (sums_ref[...].astype(jnp.float32)
                             + part).astype(jnp.bfloat16)

    @pl.when(i == _NB)
    def _loss():
        s = sums_ref[...].astype(jnp.float32)
        n2 = jnp.sum(s * s, axis=1, keepdims=True)
        inv = jnp.where(n2 > 0.0, jax.lax.rsqrt(jnp.maximum(n2, 1e-30)), 0.0)
        sn = (s * inv).astype(jnp.bfloat16)
        g = jax.lax.dot_general(
            sn, sn, (((1,), (1,)), ((), ())),
            preferred_element_type=jnp.float32)

        row_i = jax.lax.broadcasted_iota(jnp.int32, (_KP, _KP), 0)
        col_i = jax.lax.broadcasted_iota(jnp.int32, (_KP, _KP), 1)
        eye = row_i == col_i
        gd = jnp.where(eye, g, 0.0)
        zr = jnp.sum(gd, axis=1, keepdims=True)
        zc = jnp.sum(gd, axis=0, keepdims=True)
        d2 = zr + zc - 2.0 * g

        pr = (zr > 0.5).astype(jnp.float32)
        pc = (zc > 0.5).astype(jnp.float32)
        csize = jnp.sum(pr)
        ppair = pr * pc

        cm = conf_ref[...]
        omega = 0.5 * (cm + cm.T)
        omega = jnp.where(eye, 0.0, omega)

        pairmask = ppair * (row_i > col_i).astype(jnp.float32)
        contrib = pairmask * omega * jnp.exp(-_ALPHA * d2)
        num_all = jnp.sum(contrib)

        scores = jnp.where(ppair > 0.5, omega, -1.0)
        si = jnp.round(jnp.maximum(scores, 0.0) * 16777216.0).astype(jnp.int32)
        si = jnp.where(scores < 0.0, jnp.int32(-1), si)
        key_ref[...] = si * 1024 + (1023 - row_i)

        taken = jnp.int32(-(2 ** 31 - 1))

        def tk_body(_, carry):
            k = key_ref[...]
            mx = jnp.max(k, axis=0, keepdims=True)
            key_ref[...] = jnp.where(k == mx, taken, k)
            return carry

        jax.lax.fori_loop(0, _TOPM, tk_body, jnp.int32(0))
        mskf = (key_ref[...] == taken).astype(jnp.float32)
        num_tk = jnp.sum(mskf * contrib)
        den_tk = jnp.sum(mskf * pairmask)

        loss_all = num_all / jnp.maximum(csize * (csize - 1.0) * 0.5, 1.0)
        loss_tk = num_tk / jnp.maximum(den_tk, 1.0)
        loss = jnp.where(csize < 1.5, 0.0,
                         jnp.where(csize <= float(_TOPM + 1) + 0.5,
                                   loss_all, loss_tk))
        out_ref[...] = jnp.broadcast_to(loss, (1, 1))


@jax.jit
def kernel(feats, y, conf):
    y3 = y.astype(jnp.int32).reshape(_NB, 1, _BR)
    conf_p = jnp.pad(conf, ((0, _KP - _K), (0, _KP - _K)))

    out = pl.pallas_call(
        _body,
        grid=(_NB + 1,),
        in_specs=[
            pl.BlockSpec((1, 1, _BR), lambda i: (jnp.minimum(i, _NB - 1), 0, 0)),
            pl.BlockSpec((_BR, _D), lambda i: (jnp.minimum(i, _NB - 1), 0)),
            pl.BlockSpec((_KP, _KP), lambda i: (0, 0)),
        ],
        out_specs=pl.BlockSpec((1, 1), lambda i: (0, 0)),
        out_shape=jax.ShapeDtypeStruct((1, 1), jnp.float32),
        scratch_shapes=[
            pltpu.VMEM((_KP, _D), jnp.bfloat16),
            pltpu.VMEM((_KP, _KP), jnp.int32),
        ],
    )(y3, feats, conf_p)
    return out[0, 0]
